# baseline (speedup 1.0000x reference)
import functools

import jax
import jax.numpy as jnp
from jax import lax
from jax.experimental import pallas as pl
from jax.experimental.pallas import tpu as pltpu
from jax.experimental.pallas import tpu_sc as plsc

_NC = 2
_NS = 16
_NW = _NC * _NS

_B = 128
_KP = 512
_D = 512
_R = _B * _KP
_RPW = _R // _NW
_CH = 64
_NCH = _RPW // _CH

_T = 0.07
_INV_COUNT = 1.0 / (2 * _B)


def _sc_gather(mem1, mem2, idxf):
    mesh = plsc.VectorSubcoreMesh(core_axis_name="c", subcore_axis_name="s")

    @functools.partial(
        pl.kernel,
        mesh=mesh,
        compiler_params=pltpu.CompilerParams(use_tc_tiling_on_sc=False),
        out_type=(
            jax.ShapeDtypeStruct((_R, _D), jnp.float32),
            jax.ShapeDtypeStruct((_R, _D), jnp.float32),
        ),
        scratch_types=[
            pltpu.VMEM((_RPW,), jnp.int32),
            pltpu.VMEM((_CH, _D), jnp.float32),
            pltpu.VMEM((_CH, _D), jnp.float32),
            pltpu.SemaphoreType.DMA,
            pltpu.SemaphoreType.DMA,
            pltpu.SemaphoreType.DMA,
            pltpu.SemaphoreType.DMA,
        ],
    )
    def k(m1, m2, idx_hbm, out1, out2,
          idx_v, buf_a, buf_b, gsem_a, gsem_b, wsem_a, wsem_b):
        wid = lax.axis_index("s") * _NC + lax.axis_index("c")
        base = wid * _RPW

        for bank, (table, out) in enumerate(((m1, out1), (m2, out2))):
            pltpu.sync_copy(idx_hbm.at[bank, pl.ds(base, _RPW)], idx_v)

            def gather(c, buf, sem, table=table):
                c = jnp.minimum(c, _NCH - 1)
                pltpu.async_copy(table.at[idx_v.at[pl.ds(c * _CH, _CH)]],
                                 buf, sem)

            def gwait(buf, sem, table=table):
                pltpu.make_async_copy(table.at[idx_v.at[pl.ds(0, _CH)]],
                                      buf, sem).wait()

            def write(c, buf, sem, out=out):
                pltpu.async_copy(buf, out.at[pl.ds(base + c * _CH, _CH)], sem)

            def wwait(buf, sem, out=out):
                pltpu.make_async_copy(buf, out.at[pl.ds(0, _CH)], sem).wait()

            gather(0, buf_a, gsem_a)
            gather(1, buf_b, gsem_b)

            def cpair(c2, carry):
                c0 = c2 * 2
                gwait(buf_a, gsem_a)
                write(c0, buf_a, wsem_a)
                gwait(buf_b, gsem_b)
                write(c0 + 1, buf_b, wsem_b)
                wwait(buf_a, wsem_a)
                gather(c0 + 2, buf_a, gsem_a)
                wwait(buf_b, wsem_b)
                gather(c0 + 3, buf_b, gsem_b)
                return carry
            lax.fori_loop(0, _NCH // 2, cpair, 0)
            gwait(buf_a, gsem_a)
            gwait(buf_b, gsem_b)

    return k(mem1, mem2, idxf)


def _tc_loss_body(g1_ref, g2_ref, v1_ref, v2_ref, out_ref):
    b = pl.program_id(0)
    w = jnp.concatenate([g1_ref[...], g2_ref[...]], axis=0)
    vcat = jnp.concatenate([v1_ref[pl.ds(b, 1), :], v2_ref[pl.ds(b, 1), :]],
                           axis=0)
    adc = lax.dot_general(vcat, w, (((1,), (1,)), ((), ())),
                          precision=lax.Precision.HIGHEST,
                          preferred_element_type=jnp.float32) / _T
    m = jnp.max(adc, axis=1, keepdims=True)
    lse = m + jnp.log(jnp.sum(jnp.exp(adc - m), axis=1, keepdims=True))
    col = lax.broadcasted_iota(jnp.int32, adc.shape, 1)
    pos_mask = (col == 0) | (col == _KP)
    pos = jnp.sum(jnp.where(pos_mask, adc, 0.0), axis=1, keepdims=True)
    contrib = jnp.sum(pos * 0.5 - lse)
    prev = jnp.where(b == 0, 0.0, out_ref[0, 0])
    acc = prev + contrib
    out_ref[0, 0] = jnp.where(b == _B - 1, -acc * _INV_COUNT, acc)


def _tc_loss(g1, g2, v1, v2):
    out = pl.pallas_call(
        _tc_loss_body,
        grid=(_B,),
        in_specs=[
            pl.BlockSpec((_KP, _D), lambda b: (b, 0)),
            pl.BlockSpec((_KP, _D), lambda b: (b, 0)),
            pl.BlockSpec((_B, _D), lambda b: (0, 0)),
            pl.BlockSpec((_B, _D), lambda b: (0, 0)),
        ],
        out_specs=pl.BlockSpec((1, 1), lambda b: (0, 0),
                               memory_space=pltpu.SMEM),
        out_shape=jax.ShapeDtypeStruct((1, 1), jnp.float32),
    )(g1, g2, v1, v2)
    return out[0, 0]


def kernel(v1, y1, v2, y2, idx1, idx2, memory_v1, memory_v2):
    idxf = jnp.stack([idx1.reshape(-1), idx2.reshape(-1)])
    g1, g2 = _sc_gather(memory_v1, memory_v2, idxf)
    return _tc_loss(g1, g2, v1, v2)


# speedup vs baseline: 3.9277x; 3.9277x over previous
import functools

import jax
import jax.numpy as jnp
from jax import lax
from jax.experimental import pallas as pl
from jax.experimental.pallas import tpu as pltpu
from jax.experimental.pallas import tpu_sc as plsc

_NC = 2
_NS = 16
_NW = _NC * _NS

_B = 128
_KP = 512
_D = 512
_R = _B * _KP
_RPW = _R // _NW
_CH = 32
_NCH = _RPW // _CH

_T = 0.07
_INV_COUNT = 1.0 / (2 * _B)


def _sc_gather(mem1, mem2, idxf):
    mesh = plsc.VectorSubcoreMesh(core_axis_name="c", subcore_axis_name="s")

    @functools.partial(
        pl.kernel,
        mesh=mesh,
        out_type=(
            jax.ShapeDtypeStruct((_R, _D), jnp.float32),
            jax.ShapeDtypeStruct((_R, _D), jnp.float32),
        ),
        scratch_types=[
            pltpu.VMEM((_RPW,), jnp.int32),
            pltpu.VMEM((_CH, _D), jnp.float32),
            pltpu.VMEM((_CH, _D), jnp.float32),
            pltpu.SemaphoreType.DMA,
            pltpu.SemaphoreType.DMA,
            pltpu.SemaphoreType.DMA,
            pltpu.SemaphoreType.DMA,
        ],
    )
    def k(m1, m2, idx_hbm, out1, out2,
          idx_v, buf_a, buf_b, gsem_a, gsem_b, wsem_a, wsem_b):
        wid = lax.axis_index("s") * _NC + lax.axis_index("c")
        base = wid * _RPW

        for bank, (table, out) in enumerate(((m1, out1), (m2, out2))):
            pltpu.sync_copy(idx_hbm.at[bank, pl.ds(base, _RPW)], idx_v)

            def gather(c, buf, sem, table=table):
                c = jnp.minimum(c, _NCH - 1)
                pltpu.async_copy(table.at[idx_v.at[pl.ds(c * _CH, _CH)]],
                                 buf, sem)

            def gwait(buf, sem, table=table):
                pltpu.make_async_copy(table.at[idx_v.at[pl.ds(0, _CH)]],
                                      buf, sem).wait()

            def write(c, buf, sem, out=out):
                pltpu.async_copy(buf, out.at[pl.ds(base + c * _CH, _CH)], sem)

            def wwait(buf, sem, out=out):
                pltpu.make_async_copy(buf, out.at[pl.ds(0, _CH)], sem).wait()

            gather(0, buf_a, gsem_a)
            gather(1, buf_b, gsem_b)

            def cpair(c2, carry):
                c0 = c2 * 2
                gwait(buf_a, gsem_a)
                write(c0, buf_a, wsem_a)
                gwait(buf_b, gsem_b)
                write(c0 + 1, buf_b, wsem_b)
                wwait(buf_a, wsem_a)
                gather(c0 + 2, buf_a, gsem_a)
                wwait(buf_b, wsem_b)
                gather(c0 + 3, buf_b, gsem_b)
                return carry
            lax.fori_loop(0, _NCH // 2, cpair, 0)
            gwait(buf_a, gsem_a)
            gwait(buf_b, gsem_b)

    return k(mem1, mem2, idxf)


def _tc_loss_body(g1_ref, g2_ref, v1_ref, v2_ref, out_ref):
    b = pl.program_id(0)
    w = jnp.concatenate([g1_ref[...], g2_ref[...]], axis=0)
    vcat = jnp.concatenate([v1_ref[pl.ds(b, 1), :], v2_ref[pl.ds(b, 1), :]],
                           axis=0)
    adc = lax.dot_general(vcat, w, (((1,), (1,)), ((), ())),
                          precision=lax.Precision.HIGHEST,
                          preferred_element_type=jnp.float32) / _T
    m = jnp.max(adc, axis=1, keepdims=True)
    lse = m + jnp.log(jnp.sum(jnp.exp(adc - m), axis=1, keepdims=True))
    col = lax.broadcasted_iota(jnp.int32, adc.shape, 1)
    pos_mask = (col == 0) | (col == _KP)
    pos = jnp.sum(jnp.where(pos_mask, adc, 0.0), axis=1, keepdims=True)
    contrib = jnp.sum(pos * 0.5 - lse)
    prev = jnp.where(b == 0, 0.0, out_ref[0, 0])
    acc = prev + contrib
    out_ref[0, 0] = jnp.where(b == _B - 1, -acc * _INV_COUNT, acc)


def _tc_loss(g1, g2, v1, v2):
    out = pl.pallas_call(
        _tc_loss_body,
        grid=(_B,),
        in_specs=[
            pl.BlockSpec((_KP, _D), lambda b: (b, 0)),
            pl.BlockSpec((_KP, _D), lambda b: (b, 0)),
            pl.BlockSpec((_B, _D), lambda b: (0, 0)),
            pl.BlockSpec((_B, _D), lambda b: (0, 0)),
        ],
        out_specs=pl.BlockSpec((1, 1), lambda b: (0, 0),
                               memory_space=pltpu.SMEM),
        out_shape=jax.ShapeDtypeStruct((1, 1), jnp.float32),
    )(g1, g2, v1, v2)
    return out[0, 0]


def kernel(v1, y1, v2, y2, idx1, idx2, memory_v1, memory_v2):
    idxf = jnp.stack([idx1.reshape(-1), idx2.reshape(-1)])
    g1, g2 = _sc_gather(memory_v1, memory_v2, idxf)
    return _tc_loss(g1, g2, v1, v2)


# speedup vs baseline: 4.5349x; 1.1546x over previous
import functools

import jax
import jax.numpy as jnp
from jax import lax
from jax.experimental import pallas as pl
from jax.experimental.pallas import tpu as pltpu
from jax.experimental.pallas import tpu_sc as plsc

_NC = 2
_NS = 16
_NW = _NC * _NS

_B = 128
_KP = 512
_D = 512
_NF = 16084
_NFP = 16128
_R = _B * _KP
_RPW = _R // _NW
_CH = 32
_NCH = _RPW // _CH
_TPW = 2 * _B // _NW

_T = 0.07
_INV_COUNT = 1.0 / (2 * _B)


def _tc_bank1_body(vcat_ref, m1_ref, out_ref):
    out_ref[...] = lax.dot_general(
        vcat_ref[...], m1_ref[...], (((1,), (1,)), ((), ())),
        precision=lax.Precision.HIGHEST, preferred_element_type=jnp.float32)


def _tc_bank1(vcat, m1):
    return pl.pallas_call(
        _tc_bank1_body,
        grid=(_NFP // 128,),
        in_specs=[
            pl.BlockSpec((2 * _B, _D), lambda c: (0, 0)),
            pl.BlockSpec((128, _D), lambda c: (c, 0)),
        ],
        out_specs=pl.BlockSpec((2 * _B, 128), lambda c: (0, c)),
        out_shape=jax.ShapeDtypeStruct((2 * _B, _NFP), jnp.float32),
    )(vcat, m1)


def _sc_stage(mem2, idx2f, idx1f, p1t):
    mesh = plsc.VectorSubcoreMesh(core_axis_name="c", subcore_axis_name="s")

    @functools.partial(
        pl.kernel,
        mesh=mesh,
        out_type=(
            jax.ShapeDtypeStruct((_R, _D), jnp.float32),
            jax.ShapeDtypeStruct((2 * _B, _KP), jnp.float32),
        ),
        scratch_types=[
            pltpu.VMEM((_RPW,), jnp.int32),
            pltpu.VMEM((_CH, _D), jnp.float32),
            pltpu.VMEM((_CH, _D), jnp.float32),
            pltpu.VMEM((_KP,), jnp.int32),
            pltpu.VMEM((_KP,), jnp.int32),
            pltpu.VMEM((_KP,), jnp.float32),
            pltpu.SemaphoreType.DMA,
            pltpu.SemaphoreType.DMA,
            pltpu.SemaphoreType.DMA,
            pltpu.SemaphoreType.DMA,
        ],
    )
    def k(m2, i2_hbm, i1_hbm, p1t_hbm, out2, out1,
          idx_v, buf_a, buf_b, i1_v, ni_v, l1_v,
          gsem_a, gsem_b, wsem_a, wsem_b):
        wid = lax.axis_index("s") * _NC + lax.axis_index("c")

        def b1task(t, carry):
            j = wid * _TPW + t
            b = lax.rem(j, _B)
            pltpu.sync_copy(i1_hbm.at[pl.ds(b * _KP, _KP)], i1_v)

            def mkidx(c, carry2):
                ni_v[pl.ds(c * 16, 16)] = i1_v[pl.ds(c * 16, 16)] + j * _NFP
                return carry2
            lax.fori_loop(0, _KP // 16, mkidx, 0)
            for c in range(_KP // 128):
                pltpu.async_copy(
                    p1t_hbm.at[ni_v.at[pl.ds(c * 128, 128)]],
                    l1_v.at[pl.ds(c * 128, 128)], gsem_a)
            for c in range(_KP // 128):
                pltpu.make_async_copy(
                    p1t_hbm.at[ni_v.at[pl.ds(0, 128)]],
                    l1_v.at[pl.ds(0, 128)], gsem_a).wait()
            pltpu.sync_copy(l1_v, out1.at[j])
            return carry
        lax.fori_loop(0, _TPW, b1task, 0)

        base = wid * _RPW
        pltpu.sync_copy(i2_hbm.at[pl.ds(base, _RPW)], idx_v)

        def gather(c, buf, sem):
            c = jnp.minimum(c, _NCH - 1)
            pltpu.async_copy(m2.at[idx_v.at[pl.ds(c * _CH, _CH)]], buf, sem)

        def gwait(buf, sem):
            pltpu.make_async_copy(m2.at[idx_v.at[pl.ds(0, _CH)]], buf,
                                  sem).wait()

        def write(c, buf, sem):
            pltpu.async_copy(buf, out2.at[pl.ds(base + c * _CH, _CH)], sem)

        def wwait(buf, sem):
            pltpu.make_async_copy(buf, out2.at[pl.ds(0, _CH)], sem).wait()

        gather(0, buf_a, gsem_a)
        gather(1, buf_b, gsem_b)

        def cpair(c2, carry):
            c0 = c2 * 2
            gwait(buf_a, gsem_a)
            write(c0, buf_a, wsem_a)
            gwait(buf_b, gsem_b)
            write(c0 + 1, buf_b, wsem_b)
            wwait(buf_a, wsem_a)
            gather(c0 + 2, buf_a, gsem_a)
            wwait(buf_b, wsem_b)
            gather(c0 + 3, buf_b, gsem_b)
            return carry
        lax.fori_loop(0, _NCH // 2, cpair, 0)
        gwait(buf_a, gsem_a)
        gwait(buf_b, gsem_b)

    return k(mem2, idx2f, idx1f, p1t.reshape(-1))


def _tc_loss_body(g2_ref, l1_ref, v1_ref, v2_ref, out_ref):
    b = pl.program_id(0)
    vcat = jnp.concatenate([v1_ref[pl.ds(b, 1), :], v2_ref[pl.ds(b, 1), :]],
                           axis=0)
    adc2 = lax.dot_general(vcat, g2_ref[...], (((1,), (1,)), ((), ())),
                           precision=lax.Precision.HIGHEST,
                           preferred_element_type=jnp.float32)
    l1b = jnp.concatenate([l1_ref[pl.ds(b, 1), :],
                           l1_ref[pl.ds(b + _B, 1), :]], axis=0)
    adc = jnp.concatenate([l1b, adc2], axis=1) / _T
    m = jnp.max(adc, axis=1, keepdims=True)
    lse = m + jnp.log(jnp.sum(jnp.exp(adc - m), axis=1, keepdims=True))
    col = lax.broadcasted_iota(jnp.int32, adc.shape, 1)
    pos_mask = (col == 0) | (col == _KP)
    pos = jnp.sum(jnp.where(pos_mask, adc, 0.0), axis=1, keepdims=True)
    contrib = jnp.sum(pos * 0.5 - lse)
    prev = jnp.where(b == 0, 0.0, out_ref[0, 0])
    acc = prev + contrib
    out_ref[0, 0] = jnp.where(b == _B - 1, -acc * _INV_COUNT, acc)


def _tc_loss(g2, l1, v1, v2):
    out = pl.pallas_call(
        _tc_loss_body,
        grid=(_B,),
        in_specs=[
            pl.BlockSpec((_KP, _D), lambda b: (b, 0)),
            pl.BlockSpec((2 * _B, _KP), lambda b: (0, 0)),
            pl.BlockSpec((_B, _D), lambda b: (0, 0)),
            pl.BlockSpec((_B, _D), lambda b: (0, 0)),
        ],
        out_specs=pl.BlockSpec((1, 1), lambda b: (0, 0),
                               memory_space=pltpu.SMEM),
        out_shape=jax.ShapeDtypeStruct((1, 1), jnp.float32),
    )(g2, l1, v1, v2)
    return out[0, 0]


def kernel(v1, y1, v2, y2, idx1, idx2, memory_v1, memory_v2):
    vcat = jnp.concatenate([v1, v2], axis=0)
    p1t = _tc_bank1(vcat, memory_v1)
    g2, l1 = _sc_stage(memory_v2, idx2.reshape(-1), idx1.reshape(-1), p1t)
    return _tc_loss(g2, l1, v1, v2)


# speedup vs baseline: 6.8612x; 1.5130x over previous
import functools

import jax
import jax.numpy as jnp
from jax import lax
from jax.experimental import pallas as pl
from jax.experimental.pallas import tpu as pltpu
from jax.experimental.pallas import tpu_sc as plsc

_NC = 2
_NS = 16
_NW = _NC * _NS

_B = 128
_KP = 512
_D = 512
_NF = 16084
_NFP = 16128
_R = _B * _KP
_RPW = _R // _NW
_CH = 32
_NCH = _RPW // _CH
_TPW = 2 * _B // _NW

_T = 0.07
_INV_COUNT = 1.0 / (2 * _B)


def _tc_bank1_body(vcat_ref, m1_ref, out_ref):
    out_ref[...] = lax.dot_general(
        vcat_ref[...], m1_ref[...], (((1,), (1,)), ((), ())),
        precision=lax.Precision.DEFAULT, preferred_element_type=jnp.float32)


def _tc_bank1(vcat, m1):
    return pl.pallas_call(
        _tc_bank1_body,
        grid=(_NFP // 128,),
        in_specs=[
            pl.BlockSpec((2 * _B, _D), lambda c: (0, 0)),
            pl.BlockSpec((128, _D), lambda c: (c, 0)),
        ],
        out_specs=pl.BlockSpec((2 * _B, 128), lambda c: (0, c)),
        out_shape=jax.ShapeDtypeStruct((2 * _B, _NFP), jnp.float32),
    )(vcat, m1)


def _sc_stage(mem2, idx2f, idx1f, p1t):
    mesh = plsc.VectorSubcoreMesh(core_axis_name="c", subcore_axis_name="s")

    @functools.partial(
        pl.kernel,
        mesh=mesh,
        out_type=(
            jax.ShapeDtypeStruct((_R, _D), jnp.float32),
            jax.ShapeDtypeStruct((2 * _B, _KP), jnp.float32),
        ),
        scratch_types=[
            pltpu.VMEM((_RPW,), jnp.int32),
            pltpu.VMEM((_CH, _D), jnp.float32),
            pltpu.VMEM((_CH, _D), jnp.float32),
            pltpu.VMEM((_KP,), jnp.int32),
            pltpu.VMEM((_KP,), jnp.int32),
            pltpu.VMEM((_KP,), jnp.float32),
            pltpu.SemaphoreType.DMA,
            pltpu.SemaphoreType.DMA,
            pltpu.SemaphoreType.DMA,
            pltpu.SemaphoreType.DMA,
        ],
    )
    def k(m2, i2_hbm, i1_hbm, p1t_hbm, out2, out1,
          idx_v, buf_a, buf_b, i1_v, ni_v, l1_v,
          gsem_a, gsem_b, wsem_a, wsem_b):
        wid = lax.axis_index("s") * _NC + lax.axis_index("c")

        def b1task(t, carry):
            j = wid * _TPW + t
            b = lax.rem(j, _B)
            pltpu.sync_copy(i1_hbm.at[pl.ds(b * _KP, _KP)], i1_v)

            def mkidx(c, carry2):
                ni_v[pl.ds(c * 16, 16)] = i1_v[pl.ds(c * 16, 16)] + j * _NFP
                return carry2
            lax.fori_loop(0, _KP // 16, mkidx, 0)
            for c in range(_KP // 128):
                pltpu.async_copy(
                    p1t_hbm.at[ni_v.at[pl.ds(c * 128, 128)]],
                    l1_v.at[pl.ds(c * 128, 128)], gsem_a)
            for c in range(_KP // 128):
                pltpu.make_async_copy(
                    p1t_hbm.at[ni_v.at[pl.ds(0, 128)]],
                    l1_v.at[pl.ds(0, 128)], gsem_a).wait()
            pltpu.sync_copy(l1_v, out1.at[j])
            return carry
        lax.fori_loop(0, _TPW, b1task, 0)

        base = wid * _RPW
        pltpu.sync_copy(i2_hbm.at[pl.ds(base, _RPW)], idx_v)

        def gather(c, buf, sem):
            c = jnp.minimum(c, _NCH - 1)
            pltpu.async_copy(m2.at[idx_v.at[pl.ds(c * _CH, _CH)]], buf, sem)

        def gwait(buf, sem):
            pltpu.make_async_copy(m2.at[idx_v.at[pl.ds(0, _CH)]], buf,
                                  sem).wait()

        def write(c, buf, sem):
            pltpu.async_copy(buf, out2.at[pl.ds(base + c * _CH, _CH)], sem)

        def wwait(buf, sem):
            pltpu.make_async_copy(buf, out2.at[pl.ds(0, _CH)], sem).wait()

        gather(0, buf_a, gsem_a)
        gather(1, buf_b, gsem_b)

        def cpair(c2, carry):
            c0 = c2 * 2
            gwait(buf_a, gsem_a)
            write(c0, buf_a, wsem_a)
            gwait(buf_b, gsem_b)
            write(c0 + 1, buf_b, wsem_b)
            wwait(buf_a, wsem_a)
            gather(c0 + 2, buf_a, gsem_a)
            wwait(buf_b, wsem_b)
            gather(c0 + 3, buf_b, gsem_b)
            return carry
        lax.fori_loop(0, _NCH // 2, cpair, 0)
        gwait(buf_a, gsem_a)
        gwait(buf_b, gsem_b)

    return k(mem2, idx2f, idx1f, p1t.reshape(-1))


_BB = 8


def _tc_loss_body(g2_ref, l1_ref, v1_ref, v2_ref, out_ref):
    s = pl.program_id(0)
    b0 = s * _BB
    vsel = jnp.concatenate([v1_ref[pl.ds(b0, _BB), :],
                            v2_ref[pl.ds(b0, _BB), :]], axis=0)
    full = lax.dot_general(vsel, g2_ref[...], (((1,), (1,)), ((), ())),
                           precision=lax.Precision.DEFAULT,
                           preferred_element_type=jnp.float32)
    adc2 = jnp.zeros((2 * _BB, _KP), jnp.float32)
    row = lax.broadcasted_iota(jnp.int32, (2 * _BB, _KP), 0)
    for i in range(_BB):
        adc2 = jnp.where(row % _BB == i, full[:, i * _KP:(i + 1) * _KP], adc2)
    l1b = jnp.concatenate([l1_ref[pl.ds(b0, _BB), :],
                           l1_ref[pl.ds(b0 + _B, _BB), :]], axis=0)
    adc = jnp.concatenate([l1b, adc2], axis=1) / _T
    m = jnp.max(adc, axis=1, keepdims=True)
    lse = m + jnp.log(jnp.sum(jnp.exp(adc - m), axis=1, keepdims=True))
    col = lax.broadcasted_iota(jnp.int32, adc.shape, 1)
    pos_mask = (col == 0) | (col == _KP)
    pos = jnp.sum(jnp.where(pos_mask, adc, 0.0), axis=1, keepdims=True)
    contrib = jnp.sum(pos * 0.5 - lse)
    prev = jnp.where(s == 0, 0.0, out_ref[0, 0])
    acc = prev + contrib
    out_ref[0, 0] = jnp.where(s == _B // _BB - 1, -acc * _INV_COUNT, acc)


def _tc_loss(g2, l1, v1, v2):
    out = pl.pallas_call(
        _tc_loss_body,
        grid=(_B // _BB,),
        in_specs=[
            pl.BlockSpec((_BB * _KP, _D), lambda s: (s, 0)),
            pl.BlockSpec((2 * _B, _KP), lambda s: (0, 0)),
            pl.BlockSpec((_B, _D), lambda s: (0, 0)),
            pl.BlockSpec((_B, _D), lambda s: (0, 0)),
        ],
        out_specs=pl.BlockSpec((1, 1), lambda s: (0, 0),
                               memory_space=pltpu.SMEM),
        out_shape=jax.ShapeDtypeStruct((1, 1), jnp.float32),
    )(g2, l1, v1, v2)
    return out[0, 0]


def kernel(v1, y1, v2, y2, idx1, idx2, memory_v1, memory_v2):
    vcat = jnp.concatenate([v1, v2], axis=0)
    p1t = _tc_bank1(vcat, memory_v1)
    g2, l1 = _sc_stage(memory_v2, idx2.reshape(-1), idx1.reshape(-1), p1t)
    return _tc_loss(g2, l1, v1, v2)


# speedup vs baseline: 9.2003x; 1.3409x over previous
import functools

import jax
import jax.numpy as jnp
from jax import lax
from jax.experimental import pallas as pl
from jax.experimental.pallas import tpu as pltpu
from jax.experimental.pallas import tpu_sc as plsc

_NC = 2
_NS = 16
_NW = _NC * _NS

_B = 128
_KP = 512
_D = 512
_NF = 16084
_NFP = 16128
_R = _B * _KP
_RPW = _R // _NW
_CH = 32
_NCH = _RPW // _CH
_TPW = 2 * _B // _NW

_T = 0.07
_INV_COUNT = 1.0 / (2 * _B)


def _tc_bank1_body(vcat_ref, m1_ref, out_ref):
    out_ref[...] = lax.dot_general(
        vcat_ref[...], m1_ref[...], (((1,), (1,)), ((), ())),
        precision=lax.Precision.DEFAULT, preferred_element_type=jnp.float32)


def _tc_bank1(vcat, m1):
    return pl.pallas_call(
        _tc_bank1_body,
        grid=(_NFP // 1152,),
        in_specs=[
            pl.BlockSpec((2 * _B, _D), lambda c: (0, 0)),
            pl.BlockSpec((1152, _D), lambda c: (c, 0)),
        ],
        out_specs=pl.BlockSpec((2 * _B, 1152), lambda c: (0, c)),
        out_shape=jax.ShapeDtypeStruct((2 * _B, _NFP), jnp.float32),
    )(vcat, m1)


def _sc_stage(mem2, idx2f, idx1f, p1t):
    mesh = plsc.VectorSubcoreMesh(core_axis_name="c", subcore_axis_name="s")

    @functools.partial(
        pl.kernel,
        mesh=mesh,
        out_type=(
            jax.ShapeDtypeStruct((_R, _D), jnp.float32),
            jax.ShapeDtypeStruct((2 * _B, _KP), jnp.float32),
        ),
        scratch_types=[
            pltpu.VMEM((_RPW,), jnp.int32),
            pltpu.VMEM((_CH, _D), jnp.float32),
            pltpu.VMEM((_CH, _D), jnp.float32),
            pltpu.VMEM((_TPW * _KP,), jnp.int32),
            pltpu.VMEM((_TPW * _KP,), jnp.int32),
            pltpu.VMEM((_TPW, _KP), jnp.float32),
            pltpu.SemaphoreType.DMA,
            pltpu.SemaphoreType.DMA,
            pltpu.SemaphoreType.DMA,
            pltpu.SemaphoreType.DMA,
            pltpu.SemaphoreType.DMA,
        ],
    )
    def k(m2, i2_hbm, i1_hbm, p1t_hbm, out2, out1,
          idx_v, buf_a, buf_b, i1_v, ni_v, l1_v,
          gsem_a, gsem_b, wsem_a, wsem_b, psem):
        wid = lax.axis_index("s") * _NC + lax.axis_index("c")

        j0 = wid * _TPW
        b0 = lax.rem(j0, _B)
        pltpu.sync_copy(i1_hbm.at[pl.ds(b0 * _KP, _TPW * _KP)], i1_v)

        def mkidx(c, carry2):
            j = j0 + c // (_KP // 16)
            ni_v[pl.ds(c * 16, 16)] = i1_v[pl.ds(c * 16, 16)] + j * _NFP
            return carry2
        lax.fori_loop(0, _TPW * _KP // 16, mkidx, 0)
        for c in range(_TPW * _KP // 128):
            pltpu.async_copy(
                p1t_hbm.at[ni_v.at[pl.ds(c * 128, 128)]],
                l1_v.at[c // 4, pl.ds((c % 4) * 128, 128)], psem)

        base = wid * _RPW
        pltpu.sync_copy(i2_hbm.at[pl.ds(base, _RPW)], idx_v)

        def gather(c, buf, sem):
            c = jnp.minimum(c, _NCH - 1)
            pltpu.async_copy(m2.at[idx_v.at[pl.ds(c * _CH, _CH)]], buf, sem)

        def gwait(buf, sem):
            pltpu.make_async_copy(m2.at[idx_v.at[pl.ds(0, _CH)]], buf,
                                  sem).wait()

        def write(c, buf, sem):
            pltpu.async_copy(buf, out2.at[pl.ds(base + c * _CH, _CH)], sem)

        def wwait(buf, sem):
            pltpu.make_async_copy(buf, out2.at[pl.ds(0, _CH)], sem).wait()

        gather(0, buf_a, gsem_a)
        gather(1, buf_b, gsem_b)

        def cpair(c2, carry):
            c0 = c2 * 2
            gwait(buf_a, gsem_a)
            write(c0, buf_a, wsem_a)
            gwait(buf_b, gsem_b)
            write(c0 + 1, buf_b, wsem_b)
            wwait(buf_a, wsem_a)
            gather(c0 + 2, buf_a, gsem_a)
            wwait(buf_b, wsem_b)
            gather(c0 + 3, buf_b, gsem_b)
            return carry
        lax.fori_loop(0, _NCH // 2, cpair, 0)
        gwait(buf_a, gsem_a)
        gwait(buf_b, gsem_b)
        for c in range(_TPW * _KP // 128):
            pltpu.make_async_copy(
                p1t_hbm.at[ni_v.at[pl.ds(0, 128)]],
                l1_v.at[0, pl.ds(0, 128)], psem).wait()
        pltpu.sync_copy(l1_v, out1.at[pl.ds(j0, _TPW)])

    return k(mem2, idx2f, idx1f, p1t.reshape(-1))


_BB = 8


def _tc_loss_body(g2_ref, l1_ref, v1_ref, v2_ref, out_ref):
    s = pl.program_id(0)
    b0 = s * _BB
    vsel = jnp.concatenate([v1_ref[pl.ds(b0, _BB), :],
                            v2_ref[pl.ds(b0, _BB), :]], axis=0)
    full = lax.dot_general(vsel, g2_ref[...], (((1,), (1,)), ((), ())),
                           precision=lax.Precision.DEFAULT,
                           preferred_element_type=jnp.float32)
    adc2 = jnp.zeros((2 * _BB, _KP), jnp.float32)
    row = lax.broadcasted_iota(jnp.int32, (2 * _BB, _KP), 0)
    for i in range(_BB):
        adc2 = jnp.where(row % _BB == i, full[:, i * _KP:(i + 1) * _KP], adc2)
    l1b = jnp.concatenate([l1_ref[pl.ds(b0, _BB), :],
                           l1_ref[pl.ds(b0 + _B, _BB), :]], axis=0)
    adc = jnp.concatenate([l1b, adc2], axis=1) / _T
    m = jnp.max(adc, axis=1, keepdims=True)
    lse = m + jnp.log(jnp.sum(jnp.exp(adc - m), axis=1, keepdims=True))
    col = lax.broadcasted_iota(jnp.int32, adc.shape, 1)
    pos_mask = (col == 0) | (col == _KP)
    pos = jnp.sum(jnp.where(pos_mask, adc, 0.0), axis=1, keepdims=True)
    contrib = jnp.sum(pos * 0.5 - lse)
    prev = jnp.where(s == 0, 0.0, out_ref[0, 0])
    acc = prev + contrib
    out_ref[0, 0] = jnp.where(s == _B // _BB - 1, -acc * _INV_COUNT, acc)


def _tc_loss(g2, l1, v1, v2):
    out = pl.pallas_call(
        _tc_loss_body,
        grid=(_B // _BB,),
        in_specs=[
            pl.BlockSpec((_BB * _KP, _D), lambda s: (s, 0)),
            pl.BlockSpec((2 * _B, _KP), lambda s: (0, 0)),
            pl.BlockSpec((_B, _D), lambda s: (0, 0)),
            pl.BlockSpec((_B, _D), lambda s: (0, 0)),
        ],
        out_specs=pl.BlockSpec((1, 1), lambda s: (0, 0),
                               memory_space=pltpu.SMEM),
        out_shape=jax.ShapeDtypeStruct((1, 1), jnp.float32),
    )(g2, l1, v1, v2)
    return out[0, 0]


def kernel(v1, y1, v2, y2, idx1, idx2, memory_v1, memory_v2):
    vcat = jnp.concatenate([v1, v2], axis=0)
    p1t = _tc_bank1(vcat, memory_v1)
    g2, l1 = _sc_stage(memory_v2, idx2.reshape(-1), idx1.reshape(-1), p1t)
    return _tc_loss(g2, l1, v1, v2)


# speedup vs baseline: 9.6427x; 1.0481x over previous
import functools

import jax
import jax.numpy as jnp
from jax import lax
from jax.experimental import pallas as pl
from jax.experimental.pallas import tpu as pltpu
from jax.experimental.pallas import tpu_sc as plsc

_NC = 2
_NS = 16
_NW = _NC * _NS

_B = 128
_KP = 512
_D = 512
_NF = 16084
_NFP = 16128
_R = _B * _KP
_RPW = _R // _NW
_CH = 32
_NCH = _RPW // _CH
_TPW = 2 * _B // _NW

_T = 0.07
_INV_COUNT = 1.0 / (2 * _B)


def _tc_bank1_body(vcat_ref, m1_ref, out_ref):
    out_ref[...] = lax.dot_general(
        vcat_ref[...], m1_ref[...], (((1,), (1,)), ((), ())),
        precision=lax.Precision.DEFAULT, preferred_element_type=jnp.float32)


def _tc_bank1(vcat, m1):
    return pl.pallas_call(
        _tc_bank1_body,
        grid=(_NFP // 1152,),
        in_specs=[
            pl.BlockSpec((2 * _B, _D), lambda c: (0, 0)),
            pl.BlockSpec((1152, _D), lambda c: (c, 0)),
        ],
        out_specs=pl.BlockSpec((2 * _B, 1152), lambda c: (0, c)),
        out_shape=jax.ShapeDtypeStruct((2 * _B, _NFP), jnp.float32),
    )(vcat, m1)


def _sc_stage(mem2, idx2f, idx1f, p1t):
    mesh = plsc.VectorSubcoreMesh(core_axis_name="c", subcore_axis_name="s")

    @functools.partial(
        pl.kernel,
        mesh=mesh,
        out_type=(
            jax.ShapeDtypeStruct((_R, _D), jnp.float32),
            jax.ShapeDtypeStruct((2 * _B, _KP), jnp.float32),
        ),
        scratch_types=[
            pltpu.VMEM((_RPW,), jnp.int32),
            pltpu.VMEM((_CH, _D), jnp.float32),
            pltpu.VMEM((_CH, _D), jnp.float32),
            pltpu.VMEM((_CH, _D), jnp.float32),
            pltpu.VMEM((_TPW * _KP,), jnp.int32),
            pltpu.VMEM((_TPW * _KP,), jnp.int32),
            pltpu.VMEM((_TPW, _KP), jnp.float32),
            pltpu.SemaphoreType.DMA,
            pltpu.SemaphoreType.DMA,
            pltpu.SemaphoreType.DMA,
            pltpu.SemaphoreType.DMA,
            pltpu.SemaphoreType.DMA,
            pltpu.SemaphoreType.DMA,
            pltpu.SemaphoreType.DMA,
        ],
    )
    def k(m2, i2_hbm, i1_hbm, p1t_hbm, out2, out1,
          idx_v, buf_a, buf_b, buf_c, i1_v, ni_v, l1_v,
          gsem_a, gsem_b, gsem_c, wsem_a, wsem_b, wsem_c, psem):
        wid = lax.axis_index("s") * _NC + lax.axis_index("c")

        j0 = wid * _TPW
        b0 = lax.rem(j0, _B)
        pltpu.sync_copy(i1_hbm.at[pl.ds(b0 * _KP, _TPW * _KP)], i1_v)

        def mkidx(c, carry2):
            j = j0 + c // (_KP // 16)
            ni_v[pl.ds(c * 16, 16)] = i1_v[pl.ds(c * 16, 16)] + j * _NFP
            return carry2
        lax.fori_loop(0, _TPW * _KP // 16, mkidx, 0)
        for c in range(_TPW * _KP // 128):
            pltpu.async_copy(
                p1t_hbm.at[ni_v.at[pl.ds(c * 128, 128)]],
                l1_v.at[c // 4, pl.ds((c % 4) * 128, 128)], psem)

        base = wid * _RPW
        pltpu.sync_copy(i2_hbm.at[pl.ds(base, _RPW)], idx_v)

        def gather(c, buf, sem):
            c = jnp.minimum(c, _NCH - 1)
            pltpu.async_copy(m2.at[idx_v.at[pl.ds(c * _CH, _CH)]], buf, sem)

        def gwait(buf, sem):
            pltpu.make_async_copy(m2.at[idx_v.at[pl.ds(0, _CH)]], buf,
                                  sem).wait()

        def write(c, buf, sem):
            pltpu.async_copy(buf, out2.at[pl.ds(base + c * _CH, _CH)], sem)

        def wwait(buf, sem):
            pltpu.make_async_copy(buf, out2.at[pl.ds(0, _CH)], sem).wait()

        gather(0, buf_a, gsem_a)
        gather(1, buf_b, gsem_b)
        gather(2, buf_c, gsem_c)

        def ctri(c3, carry):
            c0 = c3 * 3
            gwait(buf_a, gsem_a)
            write(c0, buf_a, wsem_a)
            gwait(buf_b, gsem_b)
            write(c0 + 1, buf_b, wsem_b)
            gwait(buf_c, gsem_c)
            write(c0 + 2, buf_c, wsem_c)
            wwait(buf_a, wsem_a)
            gather(c0 + 3, buf_a, gsem_a)
            wwait(buf_b, wsem_b)
            gather(c0 + 4, buf_b, gsem_b)
            wwait(buf_c, wsem_c)
            gather(c0 + 5, buf_c, gsem_c)
            return carry
        lax.fori_loop(0, _NCH // 3, ctri, 0)
        gwait(buf_a, gsem_a)
        write(_NCH - 1, buf_a, wsem_a)
        gwait(buf_b, gsem_b)
        gwait(buf_c, gsem_c)
        wwait(buf_a, wsem_a)
        for c in range(_TPW * _KP // 128):
            pltpu.make_async_copy(
                p1t_hbm.at[ni_v.at[pl.ds(0, 128)]],
                l1_v.at[0, pl.ds(0, 128)], psem).wait()
        pltpu.sync_copy(l1_v, out1.at[pl.ds(j0, _TPW)])

    return k(mem2, idx2f, idx1f, p1t.reshape(-1))


_BB = 16


def _tc_loss_body(g2_ref, l1_ref, v1_ref, v2_ref, out_ref):
    s = pl.program_id(0)
    b0 = s * _BB
    vsel = jnp.concatenate([v1_ref[pl.ds(b0, _BB), :],
                            v2_ref[pl.ds(b0, _BB), :]], axis=0)
    full = lax.dot_general(vsel, g2_ref[...], (((1,), (1,)), ((), ())),
                           precision=lax.Precision.DEFAULT,
                           preferred_element_type=jnp.float32)
    adc2 = jnp.zeros((2 * _BB, _KP), jnp.float32)
    row = lax.broadcasted_iota(jnp.int32, (2 * _BB, _KP), 0)
    for i in range(_BB):
        adc2 = jnp.where(row % _BB == i, full[:, i * _KP:(i + 1) * _KP], adc2)
    l1b = jnp.concatenate([l1_ref[pl.ds(b0, _BB), :],
                           l1_ref[pl.ds(b0 + _B, _BB), :]], axis=0)
    adc = jnp.concatenate([l1b, adc2], axis=1) / _T
    m = jnp.max(adc, axis=1, keepdims=True)
    lse = m + jnp.log(jnp.sum(jnp.exp(adc - m), axis=1, keepdims=True))
    col = lax.broadcasted_iota(jnp.int32, adc.shape, 1)
    pos_mask = (col == 0) | (col == _KP)
    pos = jnp.sum(jnp.where(pos_mask, adc, 0.0), axis=1, keepdims=True)
    contrib = jnp.sum(pos * 0.5 - lse)
    prev = jnp.where(s == 0, 0.0, out_ref[0, 0])
    acc = prev + contrib
    out_ref[0, 0] = jnp.where(s == _B // _BB - 1, -acc * _INV_COUNT, acc)


def _tc_loss(g2, l1, v1, v2):
    out = pl.pallas_call(
        _tc_loss_body,
        grid=(_B // _BB,),
        in_specs=[
            pl.BlockSpec((_BB * _KP, _D), lambda s: (s, 0)),
            pl.BlockSpec((2 * _B, _KP), lambda s: (0, 0)),
            pl.BlockSpec((_B, _D), lambda s: (0, 0)),
            pl.BlockSpec((_B, _D), lambda s: (0, 0)),
        ],
        out_specs=pl.BlockSpec((1, 1), lambda s: (0, 0),
                               memory_space=pltpu.SMEM),
        out_shape=jax.ShapeDtypeStruct((1, 1), jnp.float32),
    )(g2, l1, v1, v2)
    return out[0, 0]


def kernel(v1, y1, v2, y2, idx1, idx2, memory_v1, memory_v2):
    vcat = jnp.concatenate([v1, v2], axis=0)
    p1t = _tc_bank1(vcat, memory_v1)
    g2, l1 = _sc_stage(memory_v2, idx2.reshape(-1), idx1.reshape(-1), p1t)
    return _tc_loss(g2, l1, v1, v2)


# speedup vs baseline: 9.7560x; 1.0117x over previous
import functools

import jax
import jax.numpy as jnp
from jax import lax
from jax.experimental import pallas as pl
from jax.experimental.pallas import tpu as pltpu
from jax.experimental.pallas import tpu_sc as plsc

_NC = 2
_NS = 16
_NW = _NC * _NS

_B = 128
_KP = 512
_D = 512
_NF = 16084
_NFP = 16128
_R = _B * _KP
_RPW = _R // _NW
_CH = 32
_NCH = _RPW // _CH
_TPW = 2 * _B // _NW

_T = 0.07
_INV_COUNT = 1.0 / (2 * _B)


def _tc_bank1_body(vcat_ref, m1_ref, out_ref):
    out_ref[...] = lax.dot_general(
        vcat_ref[...], m1_ref[...], (((1,), (1,)), ((), ())),
        precision=lax.Precision.DEFAULT, preferred_element_type=jnp.float32)


def _tc_bank1(vcat, m1):
    return pl.pallas_call(
        _tc_bank1_body,
        grid=(_NFP // 2304,),
        in_specs=[
            pl.BlockSpec((2 * _B, _D), lambda c: (0, 0)),
            pl.BlockSpec((2304, _D), lambda c: (c, 0)),
        ],
        out_specs=pl.BlockSpec((2 * _B, 2304), lambda c: (0, c)),
        out_shape=jax.ShapeDtypeStruct((2 * _B, _NFP), jnp.float32),
    )(vcat, m1)


def _sc_stage(mem2, idx2f, idx1f, p1t):
    mesh = plsc.VectorSubcoreMesh(core_axis_name="c", subcore_axis_name="s")

    @functools.partial(
        pl.kernel,
        mesh=mesh,
        out_type=(
            jax.ShapeDtypeStruct((_R, _D), jnp.float32),
            jax.ShapeDtypeStruct((2 * _B, _KP), jnp.float32),
        ),
        scratch_types=[
            pltpu.VMEM((_RPW,), jnp.int32),
            pltpu.VMEM((_CH, _D), jnp.float32),
            pltpu.VMEM((_CH, _D), jnp.float32),
            pltpu.VMEM((_CH, _D), jnp.float32),
            pltpu.VMEM((_TPW * _KP,), jnp.int32),
            pltpu.VMEM((_TPW * _KP,), jnp.int32),
            pltpu.VMEM((_TPW, _KP), jnp.float32),
            pltpu.SemaphoreType.DMA,
            pltpu.SemaphoreType.DMA,
            pltpu.SemaphoreType.DMA,
            pltpu.SemaphoreType.DMA,
            pltpu.SemaphoreType.DMA,
            pltpu.SemaphoreType.DMA,
            pltpu.SemaphoreType.DMA,
        ],
    )
    def k(m2, i2_hbm, i1_hbm, p1t_hbm, out2, out1,
          idx_v, buf_a, buf_b, buf_c, i1_v, ni_v, l1_v,
          gsem_a, gsem_b, gsem_c, wsem_a, wsem_b, wsem_c, psem):
        wid = lax.axis_index("s") * _NC + lax.axis_index("c")

        j0 = wid * _TPW
        b0 = lax.rem(j0, _B)
        pltpu.sync_copy(i1_hbm.at[pl.ds(b0 * _KP, _TPW * _KP)], i1_v)

        def mkidx(c, carry2):
            j = j0 + c // (_KP // 16)
            ni_v[pl.ds(c * 16, 16)] = i1_v[pl.ds(c * 16, 16)] + j * _NFP
            return carry2
        lax.fori_loop(0, _TPW * _KP // 16, mkidx, 0)
        for c in range(_TPW * _KP // 128):
            pltpu.async_copy(
                p1t_hbm.at[ni_v.at[pl.ds(c * 128, 128)]],
                l1_v.at[c // 4, pl.ds((c % 4) * 128, 128)], psem)

        base = wid * _RPW
        pltpu.sync_copy(i2_hbm.at[pl.ds(base, _RPW)], idx_v)

        def gather(c, buf, sem):
            c = jnp.minimum(c, _NCH - 1)
            pltpu.async_copy(m2.at[idx_v.at[pl.ds(c * _CH, _CH)]], buf, sem)

        def gwait(buf, sem):
            pltpu.make_async_copy(m2.at[idx_v.at[pl.ds(0, _CH)]], buf,
                                  sem).wait()

        def write(c, buf, sem):
            pltpu.async_copy(buf, out2.at[pl.ds(base + c * _CH, _CH)], sem)

        def wwait(buf, sem):
            pltpu.make_async_copy(buf, out2.at[pl.ds(0, _CH)], sem).wait()

        gather(0, buf_a, gsem_a)
        gather(1, buf_b, gsem_b)
        gather(2, buf_c, gsem_c)

        def ctri(c3, carry):
            c0 = c3 * 3
            gwait(buf_a, gsem_a)
            write(c0, buf_a, wsem_a)
            gwait(buf_b, gsem_b)
            write(c0 + 1, buf_b, wsem_b)
            gwait(buf_c, gsem_c)
            write(c0 + 2, buf_c, wsem_c)
            wwait(buf_a, wsem_a)
            gather(c0 + 3, buf_a, gsem_a)
            wwait(buf_b, wsem_b)
            gather(c0 + 4, buf_b, gsem_b)
            wwait(buf_c, wsem_c)
            gather(c0 + 5, buf_c, gsem_c)
            return carry
        lax.fori_loop(0, _NCH // 3, ctri, 0)
        gwait(buf_a, gsem_a)
        write(_NCH - 1, buf_a, wsem_a)
        gwait(buf_b, gsem_b)
        gwait(buf_c, gsem_c)
        wwait(buf_a, wsem_a)
        for c in range(_TPW * _KP // 128):
            pltpu.make_async_copy(
                p1t_hbm.at[ni_v.at[pl.ds(0, 128)]],
                l1_v.at[0, pl.ds(0, 128)], psem).wait()
        pltpu.sync_copy(l1_v, out1.at[pl.ds(j0, _TPW)])

    return k(mem2, idx2f, idx1f, p1t.reshape(-1))


_BB = 16


def _tc_loss_body(g2_ref, l1_ref, v1_ref, v2_ref, out_ref):
    s = pl.program_id(0)
    b0 = s * _BB
    vsel = jnp.concatenate([v1_ref[pl.ds(b0, _BB), :],
                            v2_ref[pl.ds(b0, _BB), :]], axis=0)
    full = lax.dot_general(vsel, g2_ref[...], (((1,), (1,)), ((), ())),
                           precision=lax.Precision.DEFAULT,
                           preferred_element_type=jnp.float32)
    adc2 = jnp.zeros((2 * _BB, _KP), jnp.float32)
    row = lax.broadcasted_iota(jnp.int32, (2 * _BB, _KP), 0)
    for i in range(_BB):
        adc2 = jnp.where(row % _BB == i, full[:, i * _KP:(i + 1) * _KP], adc2)
    l1b = jnp.concatenate([l1_ref[pl.ds(b0, _BB), :],
                           l1_ref[pl.ds(b0 + _B, _BB), :]], axis=0)
    adc = jnp.concatenate([l1b, adc2], axis=1) / _T
    m = jnp.max(adc, axis=1, keepdims=True)
    lse = m + jnp.log(jnp.sum(jnp.exp(adc - m), axis=1, keepdims=True))
    col = lax.broadcasted_iota(jnp.int32, adc.shape, 1)
    pos_mask = (col == 0) | (col == _KP)
    pos = jnp.sum(jnp.where(pos_mask, adc, 0.0), axis=1, keepdims=True)
    contrib = jnp.sum(pos * 0.5 - lse)
    prev = jnp.where(s == 0, 0.0, out_ref[0, 0])
    acc = prev + contrib
    out_ref[0, 0] = jnp.where(s == _B // _BB - 1, -acc * _INV_COUNT, acc)


def _tc_loss(g2, l1, v1, v2):
    out = pl.pallas_call(
        _tc_loss_body,
        grid=(_B // _BB,),
        in_specs=[
            pl.BlockSpec((_BB * _KP, _D), lambda s: (s, 0)),
            pl.BlockSpec((2 * _B, _KP), lambda s: (0, 0)),
            pl.BlockSpec((_B, _D), lambda s: (0, 0)),
            pl.BlockSpec((_B, _D), lambda s: (0, 0)),
        ],
        out_specs=pl.BlockSpec((1, 1), lambda s: (0, 0),
                               memory_space=pltpu.SMEM),
        out_shape=jax.ShapeDtypeStruct((1, 1), jnp.float32),
    )(g2, l1, v1, v2)
    return out[0, 0]


def kernel(v1, y1, v2, y2, idx1, idx2, memory_v1, memory_v2):
    vcat = jnp.concatenate([v1, v2], axis=0)
    p1t = _tc_bank1(vcat, memory_v1)
    g2, l1 = _sc_stage(memory_v2, idx2.reshape(-1), idx1.reshape(-1), p1t)
    return _tc_loss(g2, l1, v1, v2)
